# Initial kernel scaffold; baseline (speedup 1.0000x reference)
import jax
import jax.numpy as jnp
from jax.experimental import pallas as pl


def kernel(inputs):
    raise NotImplementedError("write your pallas kernel here")


# speedup vs baseline: 3.4151x; 3.4151x over previous
import jax
import jax.numpy as jnp
from jax.experimental import pallas as pl
from jax.experimental.pallas import tpu as pltpu

_K = 8
_ROWS = 128
_COLS = 32768
_LANES = 128
_STEPS = _COLS // _LANES
_R = 32


def _topk_body(x_ref, o_ref):
    init = tuple(jnp.full((_R, _LANES), -jnp.inf, jnp.float32) for _ in range(_K))

    def step(i, ts):
        cur = x_ref[:, pl.ds(i * _LANES, _LANES)]
        new = []
        for t in ts:
            hi = jnp.maximum(t, cur)
            cur = jnp.minimum(t, cur)
            new.append(hi)
        return tuple(new)

    ts = jax.lax.fori_loop(0, _STEPS, step, init)
    cand = jnp.concatenate(ts, axis=1)

    n = _K * _LANES
    iota = jax.lax.broadcasted_iota(jnp.int32, (_R, n), 1)
    big = jnp.int32(2**30)
    outs = []
    c = cand
    for _ in range(_K):
        m = jnp.max(c, axis=1, keepdims=True)
        outs.append(m)
        idx = jnp.min(jnp.where(c == m, iota, big), axis=1, keepdims=True)
        c = jnp.where(iota == idx, -jnp.inf, c)
    o_ref[...] = jnp.concatenate(outs, axis=1)


def kernel(inputs):
    grid = _ROWS // _R
    return pl.pallas_call(
        _topk_body,
        grid=(grid,),
        in_specs=[pl.BlockSpec((_R, _COLS), lambda i: (i, 0))],
        out_specs=pl.BlockSpec((_R, _K), lambda i: (i, 0)),
        out_shape=jax.ShapeDtypeStruct((_ROWS, _K), jnp.float32),
    )(inputs)
